# baseline (speedup 1.0000x reference)
import functools

import jax
import jax.numpy as jnp
from jax import lax
from jax.experimental import pallas as pl
from jax.experimental.pallas import tpu as pltpu
from jax.experimental.pallas import tpu_sc as plsc

_N = 100
_D = 50
_B = 16384
_NS = 16
_L = 16
_BPW = _B // _NS
_NPAD = 112


_HALF = _BPW // 2


def _sc_body(idx_hbm, out_hbm, idx_v, cnt_v, sem_a, sem_b):
    sid = lax.axis_index("s")
    base = sid * _BPW
    cp_a = pltpu.async_copy(
        idx_hbm.at[pl.ds(base, _HALF)], idx_v.at[pl.ds(0, _HALF)], sem_a)
    cp_b = pltpu.async_copy(
        idx_hbm.at[pl.ds(base + _HALF, _HALF)],
        idx_v.at[pl.ds(_HALF, _HALF)], sem_b)

    zero = jnp.zeros((_L,), jnp.int32)
    for g in range(_NPAD // _L):
        cnt_v[pl.ds(g * _L, _L)] = zero
    ones = zero + 1

    cp_a.wait()
    for i in range(_HALF // _L):
        xi = idx_v[pl.ds(i * _L, _L)]
        plsc.addupdate_scatter(cnt_v, [xi], ones)

    cp_b.wait()
    for i in range(_HALF // _L, _BPW // _L):
        xi = idx_v[pl.ds(i * _L, _L)]
        plsc.addupdate_scatter(cnt_v, [xi], ones)

    pltpu.sync_copy(cnt_v, out_hbm.at[sid])


_sc_counts = functools.partial(
    pl.kernel,
    mesh=plsc.VectorSubcoreMesh(
        core_axis_name="c", subcore_axis_name="s", num_cores=1),
    out_type=jax.ShapeDtypeStruct((_NS, _NPAD), jnp.int32),
    compiler_params=pltpu.CompilerParams(needs_layout_passes=False),
    scratch_types=[
        pltpu.VMEM((_BPW,), jnp.int32),
        pltpu.VMEM((_NPAD,), jnp.int32),
        pltpu.SemaphoreType.DMA,
        pltpu.SemaphoreType.DMA,
    ],
)(_sc_body)


def _reduce_body(c_ref, tab_ref, o_ref):
    counts = jnp.sum(c_ref[...], axis=0)[:_N].astype(jnp.float32)
    tab = tab_ref[...]
    norms = jnp.sqrt(jnp.sum(tab * tab, axis=1))
    o_ref[...] = jnp.sum(counts * norms).reshape(1, 1) * (1.0 / _B)


def kernel(x, entity_embeddings, relationship_embeddings):
    del relationship_embeddings
    idx = x[:, 1].astype(jnp.int32)
    counts = _sc_counts(idx)
    loss = pl.pallas_call(
        _reduce_body,
        out_shape=jax.ShapeDtypeStruct((1, 1), jnp.float32),
    )(counts, entity_embeddings)
    return loss[0, 0]


# speedup vs baseline: 1.0034x; 1.0034x over previous
import functools

import jax
import jax.numpy as jnp
from jax import lax
from jax.experimental import pallas as pl
from jax.experimental.pallas import tpu as pltpu
from jax.experimental.pallas import tpu_sc as plsc

_N = 100
_D = 50
_B = 16384
_NS = 16
_L = 16
_BPW = _B // _NS
_NPAD = 112


def _sc_body(idx_hbm, out_hbm, idx_v, cnt_v):
    sid = lax.axis_index("s")
    pltpu.sync_copy(idx_hbm.at[pl.ds(sid * _BPW, _BPW)], idx_v)

    zero = jnp.zeros((_L,), jnp.int32)
    for g in range(_NPAD // _L):
        cnt_v[pl.ds(g * _L, _L)] = zero

    ones = zero + 1
    for i in range(_BPW // _L):
        xi = idx_v[pl.ds(i * _L, _L)]
        plsc.addupdate_scatter(cnt_v, [xi], ones)

    pltpu.sync_copy(cnt_v, out_hbm.at[sid])


_sc_counts = functools.partial(
    pl.kernel,
    mesh=plsc.VectorSubcoreMesh(
        core_axis_name="c", subcore_axis_name="s", num_cores=1),
    out_type=jax.ShapeDtypeStruct((_NS, _NPAD), jnp.int32),
    compiler_params=pltpu.CompilerParams(needs_layout_passes=False),
    scratch_types=[
        pltpu.VMEM((_BPW,), jnp.int32),
        pltpu.VMEM((_NPAD,), jnp.int32),
    ],
)(_sc_body)


def _reduce_body(c_ref, tab_ref, o_ref):
    counts = jnp.sum(c_ref[...], axis=0)[:_N].astype(jnp.float32)
    tab = tab_ref[...]
    norms = jnp.sqrt(jnp.sum(tab * tab, axis=1))
    o_ref[...] = jnp.sum(counts * norms).reshape(1, 1) * (1.0 / _B)


def kernel(x, entity_embeddings, relationship_embeddings):
    del relationship_embeddings
    idx = x[:, 1].astype(jnp.int32)
    counts = _sc_counts(idx)
    loss = pl.pallas_call(
        _reduce_body,
        out_shape=jax.ShapeDtypeStruct((1, 1), jnp.float32),
    )(counts, entity_embeddings)
    return loss[0, 0]
